# baseline (speedup 1.0000x reference)
import functools
import math

import jax
import jax.numpy as jnp
from jax import lax
from jax.experimental import pallas as pl
from jax.experimental.pallas import tpu as pltpu
from jax.experimental.pallas import tpu_sc as plsc

_NC = 2
_NS = 16
_NW = _NC * _NS
_LANES = 16

_NBUF = 2


@functools.lru_cache(maxsize=None)
def _build(n_seq, seq_len, vocab, d_model, scale):
    assert n_seq % (_NW * 128) == 0 and vocab % 2 == 0
    bw = 128
    stripes = n_seq // bw
    assert stripes == _NW
    d2 = 2 * d_model
    assert d2 == 128
    groups = bw // _LANES

    mesh = plsc.VectorSubcoreMesh(core_axis_name="c", subcore_axis_name="s")

    @functools.partial(
        pl.kernel,
        mesh=mesh,
        out_type=jax.ShapeDtypeStruct((seq_len, d_model, n_seq), jnp.float32),
        scratch_types=[
            pltpu.VMEM((seq_len, bw), jnp.int32),
            [pltpu.VMEM((bw,), jnp.int32) for _ in range(_NBUF)],
            [pltpu.VMEM((bw, d2), jnp.float32) for _ in range(_NBUF)],
            [pltpu.VMEM((d_model, bw), jnp.float32) for _ in range(_NBUF)],
            [pltpu.SemaphoreType.DMA for _ in range(_NBUF)],
            [pltpu.SemaphoreType.DMA for _ in range(_NBUF)],
        ],
        compiler_params=pltpu.CompilerParams(
            needs_layout_passes=False, use_tc_tiling_on_sc=True
        ),
    )
    def emb(xt_hbm, t2_hbm, out_hbm, idx_v, pidx_v, grows_v, tbuf_v,
            gsems, osems):
        wid = lax.axis_index("s") * _NC + lax.axis_index("c")
        b0 = wid * bw

        pltpu.sync_copy(xt_hbm.at[:, pl.ds(b0, bw)], idx_v)

        lanes = lax.iota(jnp.int32, _LANES)

        def gdesc(s, b):
            return pltpu.make_async_copy(
                t2_hbm.at[pidx_v[b]], grows_v[b], gsems[b]
            )

        def wdesc(s, b):
            return pltpu.make_async_copy(
                tbuf_v[b], out_hbm.at[s, :, pl.ds(b0, bw)], osems[b]
            )

        def fire_gather(s, b):
            for g in range(groups):
                sl = pl.ds(g * _LANES, _LANES)
                pidx_v[b][sl] = idx_v[s, sl] >> 1
            gdesc(s, b).start()

        def transform(s, b):
            gdesc(s, b).wait()
            for g in range(groups):
                sl = pl.ds(g * _LANES, _LANES)
                rows = lanes + (g * _LANES)
                hcol = (idx_v[s, sl] & 1) * d_model
                for d in range(d_model):
                    v = plsc.load_gather(grows_v[b], [rows, hcol + d])
                    tbuf_v[b][d, sl] = v * scale

            wdesc(s, b).start()

        for b in range(_NBUF):
            fire_gather(b, b)

        def outer(so, carry):
            s0 = so * _NBUF
            for b in range(_NBUF):
                transform(s0 + b, b)

            @pl.when(s0 + _NBUF < seq_len)
            def _():
                for b in range(_NBUF):
                    wdesc(s0 + b, b).wait()
                    fire_gather(s0 + _NBUF + b, b)

            return carry

        lax.fori_loop(0, seq_len // _NBUF, outer, 0)

        for b in range(_NBUF):
            wdesc(seq_len - _NBUF + b, b).wait()

    return emb


def kernel(x, table):
    vocab, d_model = table.shape
    n_seq, seq_len = x.shape
    scale = float(math.sqrt(d_model))
    xt = x.T.astype(jnp.int32)
    t2 = table.reshape(vocab // 2, 2 * d_model)
    out = _build(n_seq, seq_len, vocab, d_model, scale)(xt, t2)
    return jnp.transpose(out, (2, 0, 1))


# speedup vs baseline: 1.3300x; 1.3300x over previous
import functools
import math

import jax
import jax.numpy as jnp
from jax import lax
from jax.experimental import pallas as pl
from jax.experimental.pallas import tpu as pltpu
from jax.experimental.pallas import tpu_sc as plsc

_NC = 2
_NS = 16
_NW = _NC * _NS
_LANES = 16

_NBUF = 4
_VBLK = 2000


_TBUF = 2


@functools.lru_cache(maxsize=None)
def _build_tpose(d_model, vocab):
    d2 = 2 * d_model
    nblk = (vocab // 128) * 128 // 128
    nrem = vocab - nblk * 128
    mesh = plsc.VectorSubcoreMesh(core_axis_name="c", subcore_axis_name="s")

    @functools.partial(
        pl.kernel,
        mesh=mesh,
        out_type=jax.ShapeDtypeStruct((vocab // 2, d2), jnp.float32),
        scratch_types=[
            [pltpu.VMEM((d_model, 128), jnp.float32) for _ in range(_TBUF)],
            [pltpu.VMEM((64, d2), jnp.float32) for _ in range(_TBUF)],
            pltpu.VMEM((nrem // 2, d2), jnp.float32),
            [pltpu.SemaphoreType.DMA for _ in range(_TBUF)],
            [pltpu.SemaphoreType.DMA for _ in range(_TBUF)],
            pltpu.SemaphoreType.DMA,
        ],
        compiler_params=pltpu.CompilerParams(
            needs_layout_passes=False, use_tc_tiling_on_sc=True
        ),
    )
    def tp(tt_hbm, rem_hbm, t2_hbm, blks, tbs, remv, gsems, osems, rsem):
        wid = lax.axis_index("s") * _NC + lax.axis_index("c")
        lanes = lax.iota(jnp.int32, _LANES)
        rowv = [lanes + j * _LANES for j in range(d_model // _LANES)]

        def rdesc(c, b):
            return pltpu.make_async_copy(
                tt_hbm.at[:, pl.ds(c * 128, 128)], blks[b], gsems[b]
            )

        def wdesc(c, b):
            return pltpu.make_async_copy(
                tbs[b], t2_hbm.at[pl.ds(c * 64, 64)], osems[b]
            )

        def transform(c, b):
            rdesc(c, b).wait()

            @plsc.parallel_loop(0, 64, 1, unroll=4)
            def _(p):
                for j in range(2 * d_model // _LANES):
                    col = jnp.broadcast_to(
                        2 * p + (1 if j >= d_model // _LANES else 0), (_LANES,)
                    ).astype(jnp.int32)
                    v = plsc.load_gather(
                        blks[b], [rowv[j % (d_model // _LANES)], col]
                    )
                    tbs[b][p, pl.ds(j * _LANES, _LANES)] = v

            wdesc(c, b).start()

        nmine = (nblk - wid + _NW - 1) // _NW

        def cof(k):
            return wid + k * _NW

        @pl.when(nmine > 0)
        def _():
            for b in range(_TBUF):
                @pl.when(b < nmine)
                def _():
                    rdesc(cof(b), b).start()

            def outer(k0, carry):
                for b in range(_TBUF):
                    k = k0 * _TBUF + b

                    @pl.when(k < nmine)
                    def _():
                        transform(cof(k), b)

                    @pl.when(k + _TBUF < nmine)
                    def _():
                        wdesc(cof(k), b).wait()
                        rdesc(cof(k + _TBUF), b).start()
                return carry

            lax.fori_loop(0, (nmine + _TBUF - 1) // _TBUF, outer, 0)
            for b in range(_TBUF):
                @pl.when(b < nmine)
                def _():
                    wdesc(0, b).wait()

        @pl.when(wid == 0)
        def _():
            pltpu.async_copy(rem_hbm, remv, rsem).wait()
            pltpu.async_copy(
                remv, t2_hbm.at[pl.ds(nblk * 64, nrem // 2)], rsem
            ).wait()

    return tp


@functools.lru_cache(maxsize=None)
def _build_lookup(n_seq, seq_len, vocab, d_model, scale):
    assert n_seq % (_NW * 128) == 0
    bw = 128
    d2 = 2 * d_model
    assert d2 == 128
    groups = bw // _LANES

    mesh = plsc.VectorSubcoreMesh(core_axis_name="c", subcore_axis_name="s")

    @functools.partial(
        pl.kernel,
        mesh=mesh,
        out_type=jax.ShapeDtypeStruct((seq_len, d_model, n_seq), jnp.float32),
        scratch_types=[
            pltpu.VMEM((seq_len, bw), jnp.int32),
            [pltpu.VMEM((bw,), jnp.int32) for _ in range(_NBUF)],
            [pltpu.VMEM((bw, d2), jnp.float32) for _ in range(_NBUF)],
            [pltpu.VMEM((d_model, bw), jnp.float32) for _ in range(_NBUF)],
            [pltpu.SemaphoreType.DMA for _ in range(_NBUF)],
            [pltpu.SemaphoreType.DMA for _ in range(_NBUF)],
        ],
        compiler_params=pltpu.CompilerParams(
            needs_layout_passes=False, use_tc_tiling_on_sc=True
        ),
    )
    def emb(xt_hbm, t2_hbm, out_hbm, idx_v, pidx_v, grows_v, tbuf_v,
            gsems, osems):
        wid = lax.axis_index("s") * _NC + lax.axis_index("c")
        b0 = wid * bw

        pltpu.sync_copy(xt_hbm.at[:, pl.ds(b0, bw)], idx_v)

        lanes = lax.iota(jnp.int32, _LANES)

        def gdesc(s, b):
            return pltpu.make_async_copy(
                t2_hbm.at[pidx_v[b]], grows_v[b], gsems[b]
            )

        def wdesc(s, b):
            return pltpu.make_async_copy(
                tbuf_v[b], out_hbm.at[s, :, pl.ds(b0, bw)], osems[b]
            )

        def fire_gather(s, b):
            for g in range(groups):
                sl = pl.ds(g * _LANES, _LANES)
                pidx_v[b][sl] = idx_v[s, sl] >> 1
            gdesc(s, b).start()

        def transform(s, b):
            gdesc(s, b).wait()
            for g in range(groups):
                sl = pl.ds(g * _LANES, _LANES)
                rows = lanes + (g * _LANES)
                hcol = (idx_v[s, sl] & 1) * d_model

                @plsc.parallel_loop(0, d_model, 1, unroll=8)
                def _(d):
                    v = plsc.load_gather(grows_v[b], [rows, hcol + d])
                    tbuf_v[b][d, sl] = v * scale

            wdesc(s, b).start()

        for b in range(_NBUF):
            fire_gather(b, b)

        def outer(so, carry):
            s0 = so * _NBUF
            for b in range(_NBUF):
                transform(s0 + b, b)

            @pl.when(s0 + _NBUF < seq_len)
            def _():
                for b in range(_NBUF):
                    wdesc(s0 + b, b).wait()
                    fire_gather(s0 + _NBUF + b, b)

            return carry

        lax.fori_loop(0, seq_len // _NBUF, outer, 0)

        for b in range(_NBUF):
            wdesc(seq_len - _NBUF + b, b).wait()

    return emb


def kernel(x, table):
    vocab, d_model = table.shape
    n_seq, seq_len = x.shape
    scale = float(math.sqrt(d_model))
    xt = x.T.astype(jnp.int32)
    nblk = vocab // 128
    rem2 = table[nblk * 128:].reshape(-1, 2 * d_model)
    t2 = _build_tpose(d_model, vocab)(table.T, rem2)
    out = _build_lookup(n_seq, seq_len, vocab, d_model, scale)(xt, t2)
    return jnp.transpose(out, (2, 0, 1))


# speedup vs baseline: 1.7741x; 1.3339x over previous
import functools
import math

import jax
import jax.numpy as jnp
from jax import lax
from jax.experimental import pallas as pl
from jax.experimental.pallas import tpu as pltpu
from jax.experimental.pallas import tpu_sc as plsc

_NC = 2
_NS = 16
_NW = _NC * _NS
_LANES = 16

_NBUF = 4
_LA = 2


@functools.lru_cache(maxsize=None)
def _build(n_seq, seq_len, vocab, d_model, scale):
    assert n_seq % (_NW * 128) == 0 and vocab % 2 == 0
    bw = 128
    d2 = 2 * d_model
    assert d2 == 128
    groups = bw // _LANES
    assert seq_len % _NBUF == 0

    mesh = plsc.VectorSubcoreMesh(core_axis_name="c", subcore_axis_name="s")

    @functools.partial(
        pl.kernel,
        mesh=mesh,
        out_type=jax.ShapeDtypeStruct((seq_len, d_model, n_seq), jnp.float32),
        scratch_types=[
            pltpu.VMEM((seq_len, bw), jnp.int32),
            [pltpu.VMEM((bw,), jnp.int32) for _ in range(_NBUF)],
            [pltpu.VMEM((bw, d2), jnp.float32) for _ in range(_NBUF)],
            [pltpu.VMEM((d_model, bw), jnp.float32) for _ in range(_NBUF)],
            [pltpu.SemaphoreType.DMA for _ in range(_NBUF)],
            [pltpu.SemaphoreType.DMA for _ in range(_NBUF)],
        ],
        compiler_params=pltpu.CompilerParams(needs_layout_passes=False),
    )
    def emb(xt_hbm, t2_hbm, out_hbm, idx_v, pidx_v, grows_v, tbuf_v,
            gsems, osems):
        wid = lax.axis_index("s") * _NC + lax.axis_index("c")
        b0 = wid * bw

        pltpu.sync_copy(xt_hbm.at[:, pl.ds(b0, bw)], idx_v)

        lanes = lax.iota(jnp.int32, _LANES)

        def gdesc(s, b):
            return pltpu.make_async_copy(
                t2_hbm.at[pidx_v[b]], grows_v[b], gsems[b]
            )

        def wdesc(s, b):
            return pltpu.make_async_copy(
                tbuf_v[b], out_hbm.at[s, :, pl.ds(b0, bw)], osems[b]
            )

        def fire_gather(s, b):
            for g in range(groups):
                sl = pl.ds(g * _LANES, _LANES)
                pidx_v[b][sl] = idx_v[s, sl] >> 1
            gdesc(s, b).start()

        def transform(s, b):
            gdesc(s, b).wait()
            for g in range(groups):
                sl = pl.ds(g * _LANES, _LANES)
                rows = lanes + (g * _LANES)
                hcol = (idx_v[s, sl] & 1) * d_model

                @plsc.parallel_loop(0, d_model, 1, unroll=8)
                def _(d):
                    v = plsc.load_gather(grows_v[b], [rows, hcol + d])
                    tbuf_v[b][d, sl] = v * scale

            wdesc(s, b).start()

        for s in range(_LA):
            fire_gather(s, s % _NBUF)

        def outer(so, carry):
            s0 = so * _NBUF
            for b in range(_NBUF):
                s = s0 + b

                @pl.when(s + _LA < seq_len)
                def _():
                    bla = (b + _LA) % _NBUF

                    @pl.when(s + _LA >= _NBUF)
                    def _():
                        wdesc(0, bla).wait()
                    fire_gather(s + _LA, bla)

                transform(s, b)
            return carry

        lax.fori_loop(0, seq_len // _NBUF, outer, 0)

        for j in range(seq_len - _NBUF, seq_len):
            wdesc(j, j % _NBUF).wait()

    return emb


def kernel(x, table):
    vocab, d_model = table.shape
    n_seq, seq_len = x.shape
    scale = float(math.sqrt(d_model))
    xt = x.T.astype(jnp.int32)
    t2 = table.reshape(vocab // 2, 2 * d_model)
    out = _build(n_seq, seq_len, vocab, d_model, scale)(xt, t2)
    return jnp.transpose(out, (2, 0, 1))
